# Initial kernel scaffold; baseline (speedup 1.0000x reference)
import jax
import jax.numpy as jnp
from jax.experimental import pallas as pl


def kernel(x, qkv_w, k_fc_w, k_proj_w, v_fc_w, v_proj_w, compress_mem_kv, k_pos, v_pos, strat_w, strat_b, combine_w):
    raise NotImplementedError("write your pallas kernel here")


# speedup vs baseline: 1.8160x; 1.8160x over previous
import functools

import jax
import jax.numpy as jnp
from jax.experimental import pallas as pl
from jax.experimental.pallas import tpu as pltpu

B, T, DIM = 1, 2048, 768
HEADS, DHEAD = 12, 64
HDIM = HEADS * DHEAD
CBS, SBS = 4, 4
NSEL, NMEM = 4, 1
WINDOW = 32
SCALE = 0.12
CDIM = CBS * DHEAD
HID = CDIM * 4
NBLK = T // CBS

TQ = 256
TK = 256
QT = T // TQ
CKP = 640
NEG = -1e30

f32 = jnp.float32


def _dot(a, b, precision=jax.lax.Precision.DEFAULT):
    return jax.lax.dot_general(a, b, (((1,), (0,)), ((), ())),
                               preferred_element_type=f32,
                               precision=precision)


def _dot_nt(a, b, precision=jax.lax.Precision.DEFAULT):
    return jax.lax.dot_general(a, b, (((1,), (1,)), ((), ())),
                               preferred_element_type=f32,
                               precision=precision)


def _qkv_kernel(x_ref, w3_ref, cos_ref, sin_ref, p_ref, q_ref, k_ref, v_ref):
    xb = x_ref[...]
    z = _dot(xb, w3_ref[0])
    qh = z[:, 0:DHEAD]
    kh = z[:, DHEAD:2 * DHEAD]
    vh = z[:, 2 * DHEAD:3 * DHEAD]
    cos = cos_ref[...]
    sin = sin_ref[...]
    p = p_ref[...]
    hi = jax.lax.Precision.HIGHEST
    q_ref[0] = qh * cos + _dot(qh, p, hi) * sin
    k_ref[0] = kh * cos + _dot(kh, p, hi) * sin
    v_ref[0] = vh


def _cmlp_kernel(k2_ref, v2_ref, kp_ref, vp_ref,
                 kfc_ref, kpj_ref, vfc_ref, vpj_ref, ck_ref, cv_ref):
    kin = k2_ref[0] + kp_ref[0]
    vin = v2_ref[0] + vp_ref[0]
    hk = jnp.square(jax.nn.relu(_dot(kin, kfc_ref[...])))
    ck_ref[0] = _dot(hk, kpj_ref[...])
    hv = jnp.square(jax.nn.relu(_dot(vin, vfc_ref[...])))
    cv_ref[0] = _dot(hv, vpj_ref[...])


def _cattn_kernel(q_ref, ck_ref, cv_ref, cout_ref, imp_ref):
    qt = pl.program_id(0)
    h = pl.program_id(1)
    qb = q_ref[0]
    sim = _dot_nt(qb, ck_ref[0]) * SCALE
    t = qt * TQ + jax.lax.broadcasted_iota(jnp.int32, (TQ, CKP), 0)
    s = jax.lax.broadcasted_iota(jnp.int32, (TQ, CKP), 1)
    blk_ok = (s < NBLK) & (CBS * s + (CBS - 1) < t)
    mask = blk_ok | (s == NBLK)
    sim = jnp.where(mask, sim, NEG)
    m = jnp.max(sim, axis=1, keepdims=True)
    e = jnp.exp(sim - m)
    attn = e / jnp.sum(e, axis=1, keepdims=True)
    cout_ref[0] = _dot(attn, cv_ref[0])

    @pl.when(h == 0)
    def _():
        imp_ref[...] = jnp.zeros_like(imp_ref)
    imp_ref[...] += attn[:, :NBLK] * (1.0 / HEADS)


def _topk_kernel(imp_ref, emat_ref, sel_ref):
    val = imp_ref[...]
    iota = jax.lax.broadcasted_iota(jnp.int32, (TQ, NBLK), 1)
    sel = jnp.zeros((TQ, NBLK), f32)
    for _ in range(NSEL):
        m = jnp.max(val, axis=1, keepdims=True)
        cand = jnp.where(val == m, iota, NBLK * 4)
        idx = jnp.min(cand, axis=1, keepdims=True)
        oh = iota == idx
        sel = sel + oh.astype(f32)
        val = jnp.where(oh, -1.0, val)
    sel_ref[...] = _dot(sel, emat_ref[...])


def _fw_kernel(q_ref, k_ref, v_ref, sel_ref, fout_ref, sout_ref,
               mf, lf, af, mw, lw, aw):
    qt = pl.program_id(1)
    kt = pl.program_id(2)

    @pl.when(kt == 0)
    def _():
        mf[...] = jnp.full((TQ, 1), NEG, f32)
        lf[...] = jnp.zeros((TQ, 1), f32)
        af[...] = jnp.zeros((TQ, DHEAD), f32)
        mw[...] = jnp.full((TQ, 1), NEG, f32)
        lw[...] = jnp.zeros((TQ, 1), f32)
        aw[...] = jnp.zeros((TQ, DHEAD), f32)

    @pl.when(kt <= qt)
    def _():
        qb = q_ref[0]
        kb = k_ref[0]
        vb = v_ref[0]
        sim = _dot_nt(qb, kb) * SCALE
        rows = jax.lax.broadcasted_iota(jnp.int32, (TQ, TK), 0)
        cols = jax.lax.broadcasted_iota(jnp.int32, (TQ, TK), 1)
        t = qt * TQ + rows
        s = kt * TK + cols
        causal = s <= t
        diag = (t // SBS) == (s // SBS)
        sel_exp = sel_ref[...] > 0.5
        fs = jnp.where(causal & (diag | sel_exp), sim, NEG)
        mf2 = jnp.maximum(mf[...], jnp.max(fs, axis=1, keepdims=True))
        a = jnp.exp(mf[...] - mf2)
        p = jnp.exp(fs - mf2)
        mf[...] = mf2
        lf[...] = lf[...] * a + jnp.sum(p, axis=1, keepdims=True)
        af[...] = af[...] * a + _dot(p, vb)

        @pl.when(kt + 1 >= qt)
        def _():
            dt = t - s
            ws = jnp.where((dt >= 0) & (dt < WINDOW), sim, NEG)
            mw2 = jnp.maximum(mw[...], jnp.max(ws, axis=1, keepdims=True))
            aw_ = jnp.exp(mw[...] - mw2)
            pw = jnp.exp(ws - mw2)
            mw[...] = mw2
            lw[...] = lw[...] * aw_ + jnp.sum(pw, axis=1, keepdims=True)
            aw[...] = aw[...] * aw_ + _dot(pw, vb)

    @pl.when(kt == qt)
    def _():
        fout_ref[0] = af[...] / lf[...]
        sout_ref[0] = aw[...] / lw[...]


def _combine_kernel(x_ref, wg_ref, bg_ref, cout_ref, fout_ref, sout_ref,
                    cw_ref, out_ref):
    xb = x_ref[...]
    g = jax.nn.sigmoid(_dot(xb, wg_ref[...]) + bg_ref[...])
    acc = jnp.zeros((TQ, DIM), f32)
    for h in range(HEADS):
        gc = g[:, 3 * h:3 * h + 1]
        gf = g[:, 3 * h + 1:3 * h + 2]
        gs = g[:, 3 * h + 2:3 * h + 3]
        y = gc * cout_ref[h] + gf * fout_ref[h] + gs * sout_ref[h]
        acc = acc + _dot(y, cw_ref[h])
    out_ref[...] = acc


def _build_tables():
    inv = 1.0 / (10000.0 ** (jnp.arange(0, DHEAD, 2, dtype=f32) / DHEAD))
    freqs = jnp.arange(T, dtype=f32)[:, None] * inv[None, :]
    c = jnp.cos(freqs)
    si = jnp.sin(freqs)
    cos = jnp.stack([c, c], axis=-1).reshape(T, DHEAD)
    sin = jnp.stack([-si, si], axis=-1).reshape(T, DHEAD)
    i = jnp.arange(DHEAD)
    swap = jnp.where(i % 2 == 0, i + 1, i - 1)
    p = (i[:, None] == swap[None, :]).astype(f32).T
    emat = (jnp.arange(NBLK)[:, None] ==
            (jnp.arange(T)[None, :] // CBS)).astype(f32)
    return cos, sin, p, emat


@functools.partial(jax.jit, static_argnums=())
def kernel(x, qkv_w, k_fc_w, k_proj_w, v_fc_w, v_proj_w, compress_mem_kv,
           k_pos, v_pos, strat_w, strat_b, combine_w):
    x2 = x.reshape(T, DIM)
    cos, sin, p, emat = _build_tables()

    w3 = jnp.transpose(qkv_w.reshape(3, HEADS, DHEAD, DIM), (1, 3, 0, 2))
    w3 = w3.reshape(HEADS, DIM, 3 * DHEAD)
    q, k, v = pl.pallas_call(
        _qkv_kernel,
        grid=(HEADS, QT),
        in_specs=[
            pl.BlockSpec((TQ, DIM), lambda h, qt: (qt, 0)),
            pl.BlockSpec((1, DIM, 3 * DHEAD), lambda h, qt: (h, 0, 0)),
            pl.BlockSpec((TQ, DHEAD), lambda h, qt: (qt, 0)),
            pl.BlockSpec((TQ, DHEAD), lambda h, qt: (qt, 0)),
            pl.BlockSpec((DHEAD, DHEAD), lambda h, qt: (0, 0)),
        ],
        out_specs=[
            pl.BlockSpec((1, TQ, DHEAD), lambda h, qt: (h, qt, 0)),
            pl.BlockSpec((1, TQ, DHEAD), lambda h, qt: (h, qt, 0)),
            pl.BlockSpec((1, TQ, DHEAD), lambda h, qt: (h, qt, 0)),
        ],
        out_shape=[jax.ShapeDtypeStruct((HEADS, T, DHEAD), f32)] * 3,
    )(x2, w3, cos, sin, p)

    k2 = k.reshape(HEADS, NBLK, CDIM)
    v2 = v.reshape(HEADS, NBLK, CDIM)
    kp = k_pos.reshape(HEADS, 1, CDIM)
    vp = v_pos.reshape(HEADS, 1, CDIM)
    TB = 128
    ck, cv = pl.pallas_call(
        _cmlp_kernel,
        grid=(HEADS, NBLK // TB),
        in_specs=[
            pl.BlockSpec((1, TB, CDIM), lambda h, b: (h, b, 0)),
            pl.BlockSpec((1, TB, CDIM), lambda h, b: (h, b, 0)),
            pl.BlockSpec((1, 1, CDIM), lambda h, b: (h, 0, 0)),
            pl.BlockSpec((1, 1, CDIM), lambda h, b: (h, 0, 0)),
            pl.BlockSpec((CDIM, HID), lambda h, b: (0, 0)),
            pl.BlockSpec((HID, DHEAD), lambda h, b: (0, 0)),
            pl.BlockSpec((CDIM, HID), lambda h, b: (0, 0)),
            pl.BlockSpec((HID, DHEAD), lambda h, b: (0, 0)),
        ],
        out_specs=[
            pl.BlockSpec((1, TB, DHEAD), lambda h, b: (h, b, 0)),
            pl.BlockSpec((1, TB, DHEAD), lambda h, b: (h, b, 0)),
        ],
        out_shape=[jax.ShapeDtypeStruct((HEADS, NBLK, DHEAD), f32)] * 2,
    )(k2, v2, kp, vp, k_fc_w.T, k_proj_w.T, v_fc_w.T, v_proj_w.T)

    mem_k = compress_mem_kv[0]
    mem_v = compress_mem_kv[1]
    ck_full = jnp.concatenate(
        [ck, mem_k, jnp.zeros((HEADS, CKP - NBLK - NMEM, DHEAD), f32)], axis=1)
    cv_full = jnp.concatenate(
        [cv, mem_v, jnp.zeros((HEADS, CKP - NBLK - NMEM, DHEAD), f32)], axis=1)
    cout, imp = pl.pallas_call(
        _cattn_kernel,
        grid=(QT, HEADS),
        in_specs=[
            pl.BlockSpec((1, TQ, DHEAD), lambda qt, h: (h, qt, 0)),
            pl.BlockSpec((1, CKP, DHEAD), lambda qt, h: (h, 0, 0)),
            pl.BlockSpec((1, CKP, DHEAD), lambda qt, h: (h, 0, 0)),
        ],
        out_specs=[
            pl.BlockSpec((1, TQ, DHEAD), lambda qt, h: (h, qt, 0)),
            pl.BlockSpec((TQ, NBLK), lambda qt, h: (qt, 0)),
        ],
        out_shape=[
            jax.ShapeDtypeStruct((HEADS, T, DHEAD), f32),
            jax.ShapeDtypeStruct((T, NBLK), f32),
        ],
    )(q, ck_full, cv_full)

    is_sel = pl.pallas_call(
        _topk_kernel,
        grid=(QT,),
        in_specs=[
            pl.BlockSpec((TQ, NBLK), lambda qt: (qt, 0)),
            pl.BlockSpec((NBLK, T), lambda qt: (0, 0)),
        ],
        out_specs=pl.BlockSpec((TQ, T), lambda qt: (qt, 0)),
        out_shape=jax.ShapeDtypeStruct((T, T), f32),
    )(imp, emat)

    fout, sout = pl.pallas_call(
        _fw_kernel,
        grid=(HEADS, QT, T // TK),
        in_specs=[
            pl.BlockSpec((1, TQ, DHEAD), lambda h, qt, kt: (h, qt, 0)),
            pl.BlockSpec((1, TK, DHEAD), lambda h, qt, kt: (h, kt, 0)),
            pl.BlockSpec((1, TK, DHEAD), lambda h, qt, kt: (h, kt, 0)),
            pl.BlockSpec((TQ, TK), lambda h, qt, kt: (qt, kt)),
        ],
        out_specs=[
            pl.BlockSpec((1, TQ, DHEAD), lambda h, qt, kt: (h, qt, 0)),
            pl.BlockSpec((1, TQ, DHEAD), lambda h, qt, kt: (h, qt, 0)),
        ],
        out_shape=[jax.ShapeDtypeStruct((HEADS, T, DHEAD), f32)] * 2,
        scratch_shapes=[
            pltpu.VMEM((TQ, 1), f32), pltpu.VMEM((TQ, 1), f32),
            pltpu.VMEM((TQ, DHEAD), f32),
            pltpu.VMEM((TQ, 1), f32), pltpu.VMEM((TQ, 1), f32),
            pltpu.VMEM((TQ, DHEAD), f32),
        ],
    )(q, k, v, is_sel)

    wg = jnp.concatenate([strat_w.T, jnp.zeros((DIM, 128 - 3 * HEADS), f32)],
                         axis=1)
    bg = jnp.concatenate([strat_b, jnp.zeros((128 - 3 * HEADS,), f32)])
    bg = bg.reshape(1, 128)
    cw = combine_w.T.reshape(HEADS, DHEAD, DIM)
    out = pl.pallas_call(
        _combine_kernel,
        grid=(QT,),
        in_specs=[
            pl.BlockSpec((TQ, DIM), lambda qt: (qt, 0)),
            pl.BlockSpec((DIM, 128), lambda qt: (0, 0)),
            pl.BlockSpec((1, 128), lambda qt: (0, 0)),
            pl.BlockSpec((HEADS, TQ, DHEAD), lambda qt: (0, qt, 0)),
            pl.BlockSpec((HEADS, TQ, DHEAD), lambda qt: (0, qt, 0)),
            pl.BlockSpec((HEADS, TQ, DHEAD), lambda qt: (0, qt, 0)),
            pl.BlockSpec((HEADS, DHEAD, DIM), lambda qt: (0, 0, 0)),
        ],
        out_specs=pl.BlockSpec((TQ, DIM), lambda qt: (qt, 0)),
        out_shape=jax.ShapeDtypeStruct((T, DIM), f32),
    )(x2, wg, bg, cout, fout, sout, cw)

    return out.reshape(B, T, DIM)


# speedup vs baseline: 2.7539x; 1.5165x over previous
import functools

import jax
import jax.numpy as jnp
from jax.experimental import pallas as pl
from jax.experimental.pallas import tpu as pltpu

B, T, DIM = 1, 2048, 768
HEADS, DHEAD = 12, 64
HDIM = HEADS * DHEAD
CBS, SBS = 4, 4
NSEL, NMEM = 4, 1
WINDOW = 32
SCALE = 0.12
CDIM = CBS * DHEAD
HID = CDIM * 4
NBLK = T // CBS

TQ = 256
TK = 256
QT = T // TQ
CKP = 640
NEG = -1e30

f32 = jnp.float32


def _dot(a, b, precision=jax.lax.Precision.DEFAULT):
    return jax.lax.dot_general(a, b, (((1,), (0,)), ((), ())),
                               preferred_element_type=f32,
                               precision=precision)


def _dot_nt(a, b, precision=jax.lax.Precision.DEFAULT):
    return jax.lax.dot_general(a, b, (((1,), (1,)), ((), ())),
                               preferred_element_type=f32,
                               precision=precision)


def _qkv_kernel(x_ref, w3_ref, cos_ref, sin_ref, p_ref, q_ref, k_ref, v_ref):
    h = pl.program_id(1)
    xb = x_ref[...]
    z = _dot(xb, w3_ref[h])
    qh = z[:, 0:DHEAD]
    kh = z[:, DHEAD:2 * DHEAD]
    vh = z[:, 2 * DHEAD:3 * DHEAD]
    cos = cos_ref[...]
    sin = sin_ref[...]
    p = p_ref[...]
    hi = jax.lax.Precision.HIGHEST
    q_ref[0] = qh * cos + _dot(qh, p, hi) * sin
    k_ref[0] = kh * cos + _dot(kh, p, hi) * sin
    v_ref[0] = vh


def _cmlp_kernel(k2_ref, v2_ref, kp_ref, vp_ref,
                 kfc_ref, kpj_ref, vfc_ref, vpj_ref, ck_ref, cv_ref):
    kin = k2_ref[0] + kp_ref[0]
    vin = v2_ref[0] + vp_ref[0]
    hk = jnp.square(jax.nn.relu(_dot(kin, kfc_ref[...])))
    ck_ref[0] = _dot(hk, kpj_ref[...])
    hv = jnp.square(jax.nn.relu(_dot(vin, vfc_ref[...])))
    cv_ref[0] = _dot(hv, vpj_ref[...])


def _cattn_kernel(q_ref, ck_ref, cv_ref, cout_ref, imp_ref):
    qt = pl.program_id(0)
    h = pl.program_id(1)
    qb = q_ref[0]
    sim = _dot_nt(qb, ck_ref[h]) * SCALE
    t = qt * TQ + jax.lax.broadcasted_iota(jnp.int32, (TQ, CKP), 0)
    s = jax.lax.broadcasted_iota(jnp.int32, (TQ, CKP), 1)
    blk_ok = (s < NBLK) & (CBS * s + (CBS - 1) < t)
    mask = blk_ok | (s == NBLK)
    sim = jnp.where(mask, sim, NEG)
    m = jnp.max(sim, axis=1, keepdims=True)
    e = jnp.exp(sim - m)
    attn = e / jnp.sum(e, axis=1, keepdims=True)
    cout_ref[0] = _dot(attn, cv_ref[h])

    @pl.when(h == 0)
    def _():
        imp_ref[...] = jnp.zeros_like(imp_ref)
    imp_ref[...] += attn[:, :NBLK] * (1.0 / HEADS)


def _topk_kernel(imp_ref, emat_ref, sel_ref):
    val = imp_ref[...]
    iota = jax.lax.broadcasted_iota(jnp.int32, (TQ, NBLK), 1)
    sel = jnp.zeros((TQ, NBLK), f32)
    for _ in range(NSEL):
        m = jnp.max(val, axis=1, keepdims=True)
        cand = jnp.where(val == m, iota, NBLK * 4)
        idx = jnp.min(cand, axis=1, keepdims=True)
        oh = iota == idx
        sel = sel + oh.astype(f32)
        val = jnp.where(oh, -1.0, val)
    sel_ref[...] = (_dot(sel, emat_ref[...]) > 0.5).astype(jnp.int8)


def _fw_kernel(q_ref, k_ref, v_ref, sel_ref, fout_ref, sout_ref):
    qt = pl.program_id(0)
    h = pl.program_id(1)
    qb = q_ref[0]
    rows = jax.lax.broadcasted_iota(jnp.int32, (TQ, TK), 0)
    cols = jax.lax.broadcasted_iota(jnp.int32, (TQ, TK), 1)

    def body(kt, carry):
        mf, lf, af, mw, lw, aw = carry
        kb = k_ref[h, pl.ds(kt * TK, TK), :]
        vb = v_ref[h, pl.ds(kt * TK, TK), :]
        sim = _dot_nt(qb, kb) * SCALE
        t = qt * TQ + rows
        s = kt * TK + cols
        causal = s <= t
        diag = (t // SBS) == (s // SBS)
        sel_exp = sel_ref[:, pl.ds(kt * TK, TK)] != 0
        fs = jnp.where(causal & (diag | sel_exp), sim, NEG)
        mf2 = jnp.maximum(mf, jnp.max(fs, axis=1, keepdims=True))
        a = jnp.exp(mf - mf2)
        p = jnp.exp(fs - mf2)
        lf = lf * a + jnp.sum(p, axis=1, keepdims=True)
        af = af * a + _dot(p, vb)
        dt = t - s
        ws = jnp.where((dt >= 0) & (dt < WINDOW), sim, NEG)
        mw2 = jnp.maximum(mw, jnp.max(ws, axis=1, keepdims=True))
        aw_ = jnp.exp(mw - mw2)
        pw = jnp.exp(ws - mw2)
        lw = lw * aw_ + jnp.sum(pw, axis=1, keepdims=True)
        aw = aw * aw_ + _dot(pw, vb)
        return mf2, lf, af, mw2, lw, aw

    init = (jnp.full((TQ, 1), NEG, f32), jnp.zeros((TQ, 1), f32),
            jnp.zeros((TQ, DHEAD), f32),
            jnp.full((TQ, 1), NEG, f32), jnp.zeros((TQ, 1), f32),
            jnp.zeros((TQ, DHEAD), f32))
    mf, lf, af, mw, lw, aw = jax.lax.fori_loop(0, qt + 1, body, init)
    fout_ref[0] = af / lf
    sout_ref[0] = aw / lw


def _combine_kernel(x_ref, wg_ref, bg_ref, cout_ref, fout_ref, sout_ref,
                    cw_ref, out_ref):
    xb = x_ref[...]
    g = jax.nn.sigmoid(_dot(xb, wg_ref[...]) + bg_ref[...])
    acc = jnp.zeros((TQ, DIM), f32)
    for h in range(HEADS):
        gc = g[:, 3 * h:3 * h + 1]
        gf = g[:, 3 * h + 1:3 * h + 2]
        gs = g[:, 3 * h + 2:3 * h + 3]
        y = gc * cout_ref[h] + gf * fout_ref[h] + gs * sout_ref[h]
        acc = acc + _dot(y, cw_ref[h])
    out_ref[...] = acc


def _build_tables():
    inv = 1.0 / (10000.0 ** (jnp.arange(0, DHEAD, 2, dtype=f32) / DHEAD))
    freqs = jnp.arange(T, dtype=f32)[:, None] * inv[None, :]
    c = jnp.cos(freqs)
    si = jnp.sin(freqs)
    cos = jnp.stack([c, c], axis=-1).reshape(T, DHEAD)
    sin = jnp.stack([-si, si], axis=-1).reshape(T, DHEAD)
    i = jnp.arange(DHEAD)
    swap = jnp.where(i % 2 == 0, i + 1, i - 1)
    p = (i[:, None] == swap[None, :]).astype(f32).T
    emat = (jnp.arange(NBLK)[:, None] ==
            (jnp.arange(T)[None, :] // CBS)).astype(f32)
    return cos, sin, p, emat


@functools.partial(jax.jit, static_argnums=())
def kernel(x, qkv_w, k_fc_w, k_proj_w, v_fc_w, v_proj_w, compress_mem_kv,
           k_pos, v_pos, strat_w, strat_b, combine_w):
    x2 = x.reshape(T, DIM)
    cos, sin, p, emat = _build_tables()

    w3 = jnp.transpose(qkv_w.reshape(3, HEADS, DHEAD, DIM), (1, 3, 0, 2))
    w3 = w3.reshape(HEADS, DIM, 3 * DHEAD)
    q, k, v = pl.pallas_call(
        _qkv_kernel,
        grid=(QT, HEADS),
        in_specs=[
            pl.BlockSpec((TQ, DIM), lambda qt, h: (qt, 0)),
            pl.BlockSpec((HEADS, DIM, 3 * DHEAD), lambda qt, h: (0, 0, 0)),
            pl.BlockSpec((TQ, DHEAD), lambda qt, h: (qt, 0)),
            pl.BlockSpec((TQ, DHEAD), lambda qt, h: (qt, 0)),
            pl.BlockSpec((DHEAD, DHEAD), lambda qt, h: (0, 0)),
        ],
        out_specs=[
            pl.BlockSpec((1, TQ, DHEAD), lambda qt, h: (h, qt, 0)),
            pl.BlockSpec((1, TQ, DHEAD), lambda qt, h: (h, qt, 0)),
            pl.BlockSpec((1, TQ, DHEAD), lambda qt, h: (h, qt, 0)),
        ],
        out_shape=[jax.ShapeDtypeStruct((HEADS, T, DHEAD), f32)] * 3,
    )(x2, w3, cos, sin, p)

    k2 = k.reshape(HEADS, NBLK, CDIM)
    v2 = v.reshape(HEADS, NBLK, CDIM)
    kp = k_pos.reshape(HEADS, 1, CDIM)
    vp = v_pos.reshape(HEADS, 1, CDIM)
    TB = 128
    ck, cv = pl.pallas_call(
        _cmlp_kernel,
        grid=(HEADS, NBLK // TB),
        in_specs=[
            pl.BlockSpec((1, TB, CDIM), lambda h, b: (h, b, 0)),
            pl.BlockSpec((1, TB, CDIM), lambda h, b: (h, b, 0)),
            pl.BlockSpec((1, 1, CDIM), lambda h, b: (h, 0, 0)),
            pl.BlockSpec((1, 1, CDIM), lambda h, b: (h, 0, 0)),
            pl.BlockSpec((CDIM, HID), lambda h, b: (0, 0)),
            pl.BlockSpec((HID, DHEAD), lambda h, b: (0, 0)),
            pl.BlockSpec((CDIM, HID), lambda h, b: (0, 0)),
            pl.BlockSpec((HID, DHEAD), lambda h, b: (0, 0)),
        ],
        out_specs=[
            pl.BlockSpec((1, TB, DHEAD), lambda h, b: (h, b, 0)),
            pl.BlockSpec((1, TB, DHEAD), lambda h, b: (h, b, 0)),
        ],
        out_shape=[jax.ShapeDtypeStruct((HEADS, NBLK, DHEAD), f32)] * 2,
    )(k2, v2, kp, vp, k_fc_w.T, k_proj_w.T, v_fc_w.T, v_proj_w.T)

    mem_k = compress_mem_kv[0]
    mem_v = compress_mem_kv[1]
    ck_full = jnp.concatenate(
        [ck, mem_k, jnp.zeros((HEADS, CKP - NBLK - NMEM, DHEAD), f32)], axis=1)
    cv_full = jnp.concatenate(
        [cv, mem_v, jnp.zeros((HEADS, CKP - NBLK - NMEM, DHEAD), f32)], axis=1)
    cout, imp = pl.pallas_call(
        _cattn_kernel,
        grid=(QT, HEADS),
        in_specs=[
            pl.BlockSpec((1, TQ, DHEAD), lambda qt, h: (h, qt, 0)),
            pl.BlockSpec((HEADS, CKP, DHEAD), lambda qt, h: (0, 0, 0)),
            pl.BlockSpec((HEADS, CKP, DHEAD), lambda qt, h: (0, 0, 0)),
        ],
        out_specs=[
            pl.BlockSpec((1, TQ, DHEAD), lambda qt, h: (h, qt, 0)),
            pl.BlockSpec((TQ, NBLK), lambda qt, h: (qt, 0)),
        ],
        out_shape=[
            jax.ShapeDtypeStruct((HEADS, T, DHEAD), f32),
            jax.ShapeDtypeStruct((T, NBLK), f32),
        ],
    )(q, ck_full, cv_full)

    is_sel = pl.pallas_call(
        _topk_kernel,
        grid=(QT,),
        in_specs=[
            pl.BlockSpec((TQ, NBLK), lambda qt: (qt, 0)),
            pl.BlockSpec((NBLK, T), lambda qt: (0, 0)),
        ],
        out_specs=pl.BlockSpec((TQ, T), lambda qt: (qt, 0)),
        out_shape=jax.ShapeDtypeStruct((T, T), jnp.int8),
    )(imp, emat)

    fout, sout = pl.pallas_call(
        _fw_kernel,
        grid=(QT, HEADS),
        in_specs=[
            pl.BlockSpec((1, TQ, DHEAD), lambda qt, h: (h, qt, 0)),
            pl.BlockSpec((HEADS, T, DHEAD), lambda qt, h: (0, 0, 0)),
            pl.BlockSpec((HEADS, T, DHEAD), lambda qt, h: (0, 0, 0)),
            pl.BlockSpec((TQ, T), lambda qt, h: (qt, 0)),
        ],
        out_specs=[
            pl.BlockSpec((1, TQ, DHEAD), lambda qt, h: (h, qt, 0)),
            pl.BlockSpec((1, TQ, DHEAD), lambda qt, h: (h, qt, 0)),
        ],
        out_shape=[jax.ShapeDtypeStruct((HEADS, T, DHEAD), f32)] * 2,
    )(q, k, v, is_sel)

    wg = jnp.concatenate([strat_w.T, jnp.zeros((DIM, 128 - 3 * HEADS), f32)],
                         axis=1)
    bg = jnp.concatenate([strat_b, jnp.zeros((128 - 3 * HEADS,), f32)])
    bg = bg.reshape(1, 128)
    cw = combine_w.T.reshape(HEADS, DHEAD, DIM)
    out = pl.pallas_call(
        _combine_kernel,
        grid=(QT,),
        in_specs=[
            pl.BlockSpec((TQ, DIM), lambda qt: (qt, 0)),
            pl.BlockSpec((DIM, 128), lambda qt: (0, 0)),
            pl.BlockSpec((1, 128), lambda qt: (0, 0)),
            pl.BlockSpec((HEADS, TQ, DHEAD), lambda qt: (0, qt, 0)),
            pl.BlockSpec((HEADS, TQ, DHEAD), lambda qt: (0, qt, 0)),
            pl.BlockSpec((HEADS, TQ, DHEAD), lambda qt: (0, qt, 0)),
            pl.BlockSpec((HEADS, DHEAD, DIM), lambda qt: (0, 0, 0)),
        ],
        out_specs=pl.BlockSpec((TQ, DIM), lambda qt: (qt, 0)),
        out_shape=jax.ShapeDtypeStruct((T, DIM), f32),
    )(x2, wg, bg, cout, fout, sout, cw)

    return out.reshape(B, T, DIM)


# speedup vs baseline: 3.1125x; 1.1302x over previous
import functools

import jax
import jax.numpy as jnp
from jax.experimental import pallas as pl
from jax.experimental.pallas import tpu as pltpu

B, T, DIM = 1, 2048, 768
HEADS, DHEAD = 12, 64
HDIM = HEADS * DHEAD
CBS, SBS = 4, 4
NSEL, NMEM = 4, 1
WINDOW = 32
SCALE = 0.12
CDIM = CBS * DHEAD
HID = CDIM * 4
NBLK = T // CBS

TQ = 256
TK = 256
QT = T // TQ
CKP = 640
NEG = -1e30

f32 = jnp.float32


def _dot(a, b, precision=jax.lax.Precision.DEFAULT):
    return jax.lax.dot_general(a, b, (((1,), (0,)), ((), ())),
                               preferred_element_type=f32,
                               precision=precision)


def _dot_nt(a, b, precision=jax.lax.Precision.DEFAULT):
    return jax.lax.dot_general(a, b, (((1,), (1,)), ((), ())),
                               preferred_element_type=f32,
                               precision=precision)


def _qkv_kernel(x_ref, w3_ref, cos_ref, sin_ref, p_ref,
                q_ref, k_ref, kb_ref, v_ref, vb_ref):
    h = pl.program_id(1)
    xb = x_ref[...]
    z = _dot(xb, w3_ref[h])
    qh = z[:, 0:DHEAD]
    kh = z[:, DHEAD:2 * DHEAD]
    vh = z[:, 2 * DHEAD:3 * DHEAD]
    cos = cos_ref[...]
    sin = sin_ref[...]
    p = p_ref[...]
    hi = jax.lax.Precision.HIGHEST
    qr = qh * cos + _dot(qh, p, hi) * sin
    kr = kh * cos + _dot(kh, p, hi) * sin
    q_ref[0] = qr.astype(jnp.bfloat16)
    k_ref[0] = kr
    kb_ref[0] = kr.astype(jnp.bfloat16)
    v_ref[0] = vh
    vb_ref[0] = vh.astype(jnp.bfloat16)


def _cmlp_kernel(k2_ref, v2_ref, kp_ref, vp_ref,
                 kfc_ref, kpj_ref, vfc_ref, vpj_ref, ck_ref, cv_ref):
    kin = k2_ref[0] + kp_ref[0]
    vin = v2_ref[0] + vp_ref[0]
    hk = jnp.square(jax.nn.relu(_dot(kin, kfc_ref[...])))
    ck_ref[0] = _dot(hk, kpj_ref[...]).astype(jnp.bfloat16)
    hv = jnp.square(jax.nn.relu(_dot(vin, vfc_ref[...])))
    cv_ref[0] = _dot(hv, vpj_ref[...]).astype(jnp.bfloat16)


def _cattn_kernel(q_ref, ck_ref, cv_ref, cout_ref, imp_ref):
    qt = pl.program_id(0)
    h = pl.program_id(1)
    qb = q_ref[0]
    sim = _dot_nt(qb, ck_ref[h]) * SCALE
    t = qt * TQ + jax.lax.broadcasted_iota(jnp.int32, (TQ, CKP), 0)
    s = jax.lax.broadcasted_iota(jnp.int32, (TQ, CKP), 1)
    blk_ok = (s < NBLK) & (CBS * s + (CBS - 1) < t)
    mask = blk_ok | (s == NBLK)
    sim = jnp.where(mask, sim, NEG)
    m = jnp.max(sim, axis=1, keepdims=True)
    e = jnp.exp(sim - m)
    attn = e / jnp.sum(e, axis=1, keepdims=True)
    cout_ref[0] = _dot(attn, cv_ref[h])

    @pl.when(h == 0)
    def _():
        imp_ref[...] = jnp.zeros_like(imp_ref)
    imp_ref[...] += attn[:, :NBLK] * (1.0 / HEADS)


def _topk_kernel(imp_ref, emat_ref, sel_ref):
    val = imp_ref[...]
    iota = jax.lax.broadcasted_iota(jnp.int32, (TQ, NBLK), 1)
    sel = jnp.zeros((TQ, NBLK), f32)
    for _ in range(NSEL):
        m = jnp.max(val, axis=1, keepdims=True)
        cand = jnp.where(val == m, iota, NBLK * 4)
        idx = jnp.min(cand, axis=1, keepdims=True)
        oh = iota == idx
        sel = sel + oh.astype(f32)
        val = jnp.where(oh, -1.0, val)
    sel_ref[...] = (_dot(sel, emat_ref[...]) > 0.5).astype(jnp.int8)


WPAD = 32
WCOLS = 320


def _fw_kernel(q_ref, kw_ref, vw_ref, sel_ref, fout_ref, sout_ref):
    qt = pl.program_id(0)
    h = pl.program_id(1)
    qb = q_ref[0]

    def body(kt, carry):
        mf, lf, af = carry
        kb = kw_ref[h, pl.ds(kt * TK + WPAD, TK), :]
        vb = vw_ref[h, pl.ds(kt * TK + WPAD, TK), :]
        sim = _dot_nt(qb, kb) * SCALE
        fs = jnp.where(sel_ref[:, pl.ds(kt * TK, TK)] != 0, sim, NEG)
        mf2 = jnp.maximum(mf, jnp.max(fs, axis=1, keepdims=True))
        a = jnp.exp(mf - mf2)
        p = jnp.exp(fs - mf2)
        lf = lf * a + jnp.sum(p, axis=1, keepdims=True)
        af = af * a + _dot(p, vb)
        return mf2, lf, af

    init = (jnp.full((TQ, 1), NEG, f32), jnp.zeros((TQ, 1), f32),
            jnp.zeros((TQ, DHEAD), f32))
    mf, lf, af = jax.lax.fori_loop(0, qt, body, init)

    kb = kw_ref[h, pl.ds(qt * TK + WPAD, TK), :]
    vb = vw_ref[h, pl.ds(qt * TK + WPAD, TK), :]
    sim = _dot_nt(qb, kb) * SCALE
    r = jax.lax.broadcasted_iota(jnp.int32, (TQ, TK), 0)
    c = jax.lax.broadcasted_iota(jnp.int32, (TQ, TK), 1)
    dmask = (c <= r) & (((r // SBS) == (c // SBS)) |
                        (sel_ref[:, pl.ds(qt * TK, TK)] != 0))
    fs = jnp.where(dmask, sim, NEG)
    mf2 = jnp.maximum(mf, jnp.max(fs, axis=1, keepdims=True))
    a = jnp.exp(mf - mf2)
    p = jnp.exp(fs - mf2)
    lf = lf * a + jnp.sum(p, axis=1, keepdims=True)
    af = af * a + _dot(p, vb)
    fout_ref[0] = af / lf

    kb = kw_ref[h, pl.ds(qt * TK, WCOLS), :]
    vb = vw_ref[h, pl.ds(qt * TK, WCOLS), :]
    ws = _dot_nt(qb, kb) * SCALE
    rw = jax.lax.broadcasted_iota(jnp.int32, (TQ, WCOLS), 0)
    cw = jax.lax.broadcasted_iota(jnp.int32, (TQ, WCOLS), 1)
    dt = rw + WPAD - cw
    s_ok = (qt * TK - WPAD + cw) >= 0
    wmask = (dt >= 0) & (dt < WINDOW) & s_ok
    ws = jnp.where(wmask, ws, NEG)
    mw = jnp.max(ws, axis=1, keepdims=True)
    pw = jnp.exp(ws - mw)
    sout_ref[0] = _dot(pw, vb) / jnp.sum(pw, axis=1, keepdims=True)


def _combine_kernel(x_ref, wg_ref, bg_ref, cout_ref, fout_ref, sout_ref,
                    cw_ref, out_ref):
    xb = x_ref[...]
    g = jax.nn.sigmoid(_dot(xb, wg_ref[...]) + bg_ref[...])
    acc = jnp.zeros((TQ, DIM), f32)
    for h in range(HEADS):
        gc = g[:, 3 * h:3 * h + 1]
        gf = g[:, 3 * h + 1:3 * h + 2]
        gs = g[:, 3 * h + 2:3 * h + 3]
        y = gc * cout_ref[h] + gf * fout_ref[h] + gs * sout_ref[h]
        acc = acc + _dot(y, cw_ref[h])
    out_ref[...] = acc


def _build_tables():
    inv = 1.0 / (10000.0 ** (jnp.arange(0, DHEAD, 2, dtype=f32) / DHEAD))
    freqs = jnp.arange(T, dtype=f32)[:, None] * inv[None, :]
    c = jnp.cos(freqs)
    si = jnp.sin(freqs)
    cos = jnp.stack([c, c], axis=-1).reshape(T, DHEAD)
    sin = jnp.stack([-si, si], axis=-1).reshape(T, DHEAD)
    i = jnp.arange(DHEAD)
    swap = jnp.where(i % 2 == 0, i + 1, i - 1)
    p = (i[:, None] == swap[None, :]).astype(f32).T
    emat = (jnp.arange(NBLK)[:, None] ==
            (jnp.arange(T)[None, :] // CBS)).astype(f32)
    return cos, sin, p, emat


@functools.partial(jax.jit, static_argnums=())
def kernel(x, qkv_w, k_fc_w, k_proj_w, v_fc_w, v_proj_w, compress_mem_kv,
           k_pos, v_pos, strat_w, strat_b, combine_w):
    x2 = x.reshape(T, DIM)
    cos, sin, p, emat = _build_tables()

    w3 = jnp.transpose(qkv_w.reshape(3, HEADS, DHEAD, DIM), (1, 3, 0, 2))
    w3 = w3.reshape(HEADS, DIM, 3 * DHEAD)
    q, k, kb16, v, vb16 = pl.pallas_call(
        _qkv_kernel,
        grid=(QT, HEADS),
        in_specs=[
            pl.BlockSpec((TQ, DIM), lambda qt, h: (qt, 0)),
            pl.BlockSpec((HEADS, DIM, 3 * DHEAD), lambda qt, h: (0, 0, 0)),
            pl.BlockSpec((TQ, DHEAD), lambda qt, h: (qt, 0)),
            pl.BlockSpec((TQ, DHEAD), lambda qt, h: (qt, 0)),
            pl.BlockSpec((DHEAD, DHEAD), lambda qt, h: (0, 0)),
        ],
        out_specs=[
            pl.BlockSpec((1, TQ, DHEAD), lambda qt, h: (h, qt, 0)),
            pl.BlockSpec((1, TQ, DHEAD), lambda qt, h: (h, qt, 0)),
            pl.BlockSpec((1, TQ, DHEAD), lambda qt, h: (h, qt, 0)),
            pl.BlockSpec((1, TQ, DHEAD), lambda qt, h: (h, qt, 0)),
            pl.BlockSpec((1, TQ, DHEAD), lambda qt, h: (h, qt, 0)),
        ],
        out_shape=[
            jax.ShapeDtypeStruct((HEADS, T, DHEAD), jnp.bfloat16),
            jax.ShapeDtypeStruct((HEADS, T, DHEAD), f32),
            jax.ShapeDtypeStruct((HEADS, T, DHEAD), jnp.bfloat16),
            jax.ShapeDtypeStruct((HEADS, T, DHEAD), f32),
            jax.ShapeDtypeStruct((HEADS, T, DHEAD), jnp.bfloat16),
        ],
    )(x2, w3, cos, sin, p)

    k2 = k.reshape(HEADS, NBLK, CDIM)
    v2 = v.reshape(HEADS, NBLK, CDIM)
    kp = k_pos.reshape(HEADS, 1, CDIM)
    vp = v_pos.reshape(HEADS, 1, CDIM)
    TB = 128
    ck, cv = pl.pallas_call(
        _cmlp_kernel,
        grid=(HEADS, NBLK // TB),
        in_specs=[
            pl.BlockSpec((1, TB, CDIM), lambda h, b: (h, b, 0)),
            pl.BlockSpec((1, TB, CDIM), lambda h, b: (h, b, 0)),
            pl.BlockSpec((1, 1, CDIM), lambda h, b: (h, 0, 0)),
            pl.BlockSpec((1, 1, CDIM), lambda h, b: (h, 0, 0)),
            pl.BlockSpec((CDIM, HID), lambda h, b: (0, 0)),
            pl.BlockSpec((HID, DHEAD), lambda h, b: (0, 0)),
            pl.BlockSpec((CDIM, HID), lambda h, b: (0, 0)),
            pl.BlockSpec((HID, DHEAD), lambda h, b: (0, 0)),
        ],
        out_specs=[
            pl.BlockSpec((1, TB, DHEAD), lambda h, b: (h, b, 0)),
            pl.BlockSpec((1, TB, DHEAD), lambda h, b: (h, b, 0)),
        ],
        out_shape=[jax.ShapeDtypeStruct((HEADS, NBLK, DHEAD),
                                        jnp.bfloat16)] * 2,
    )(k2, v2, kp, vp, k_fc_w.T, k_proj_w.T, v_fc_w.T, v_proj_w.T)

    bf16 = jnp.bfloat16
    mem_k = compress_mem_kv[0].astype(bf16)
    mem_v = compress_mem_kv[1].astype(bf16)
    zpad = jnp.zeros((HEADS, CKP - NBLK - NMEM, DHEAD), bf16)
    ck_full = jnp.concatenate([ck, mem_k, zpad], axis=1)
    cv_full = jnp.concatenate([cv, mem_v, zpad], axis=1)
    cout, imp = pl.pallas_call(
        _cattn_kernel,
        grid=(QT, HEADS),
        in_specs=[
            pl.BlockSpec((1, TQ, DHEAD), lambda qt, h: (h, qt, 0)),
            pl.BlockSpec((HEADS, CKP, DHEAD), lambda qt, h: (0, 0, 0)),
            pl.BlockSpec((HEADS, CKP, DHEAD), lambda qt, h: (0, 0, 0)),
        ],
        out_specs=[
            pl.BlockSpec((1, TQ, DHEAD), lambda qt, h: (h, qt, 0)),
            pl.BlockSpec((TQ, NBLK), lambda qt, h: (qt, 0)),
        ],
        out_shape=[
            jax.ShapeDtypeStruct((HEADS, T, DHEAD), f32),
            jax.ShapeDtypeStruct((T, NBLK), f32),
        ],
    )(q, ck_full, cv_full)

    is_sel = pl.pallas_call(
        _topk_kernel,
        grid=(QT,),
        in_specs=[
            pl.BlockSpec((TQ, NBLK), lambda qt: (qt, 0)),
            pl.BlockSpec((NBLK, T), lambda qt: (0, 0)),
        ],
        out_specs=pl.BlockSpec((TQ, T), lambda qt: (qt, 0)),
        out_shape=jax.ShapeDtypeStruct((T, T), jnp.int8),
    )(imp, emat)

    TW = WPAD + T + (WCOLS - TK - WPAD)
    kw = jnp.concatenate(
        [jnp.zeros((HEADS, WPAD, DHEAD), bf16), kb16,
         jnp.zeros((HEADS, TW - WPAD - T, DHEAD), bf16)], axis=1)
    vw = jnp.concatenate(
        [jnp.zeros((HEADS, WPAD, DHEAD), bf16), vb16,
         jnp.zeros((HEADS, TW - WPAD - T, DHEAD), bf16)], axis=1)
    fout, sout = pl.pallas_call(
        _fw_kernel,
        grid=(QT, HEADS),
        in_specs=[
            pl.BlockSpec((1, TQ, DHEAD), lambda qt, h: (h, qt, 0)),
            pl.BlockSpec((HEADS, TW, DHEAD), lambda qt, h: (0, 0, 0)),
            pl.BlockSpec((HEADS, TW, DHEAD), lambda qt, h: (0, 0, 0)),
            pl.BlockSpec((TQ, T), lambda qt, h: (qt, 0)),
        ],
        out_specs=[
            pl.BlockSpec((1, TQ, DHEAD), lambda qt, h: (h, qt, 0)),
            pl.BlockSpec((1, TQ, DHEAD), lambda qt, h: (h, qt, 0)),
        ],
        out_shape=[jax.ShapeDtypeStruct((HEADS, T, DHEAD), f32)] * 2,
    )(q, kw, vw, is_sel)

    wg = jnp.concatenate([strat_w.T, jnp.zeros((DIM, 128 - 3 * HEADS), f32)],
                         axis=1)
    bg = jnp.concatenate([strat_b, jnp.zeros((128 - 3 * HEADS,), f32)])
    bg = bg.reshape(1, 128)
    cw = combine_w.T.reshape(HEADS, DHEAD, DIM)
    out = pl.pallas_call(
        _combine_kernel,
        grid=(QT,),
        in_specs=[
            pl.BlockSpec((TQ, DIM), lambda qt: (qt, 0)),
            pl.BlockSpec((DIM, 128), lambda qt: (0, 0)),
            pl.BlockSpec((1, 128), lambda qt: (0, 0)),
            pl.BlockSpec((HEADS, TQ, DHEAD), lambda qt: (0, qt, 0)),
            pl.BlockSpec((HEADS, TQ, DHEAD), lambda qt: (0, qt, 0)),
            pl.BlockSpec((HEADS, TQ, DHEAD), lambda qt: (0, qt, 0)),
            pl.BlockSpec((HEADS, DHEAD, DIM), lambda qt: (0, 0, 0)),
        ],
        out_specs=pl.BlockSpec((TQ, DIM), lambda qt: (qt, 0)),
        out_shape=jax.ShapeDtypeStruct((T, DIM), f32),
    )(x2, wg, bg, cout, fout, sout, cw)

    return out.reshape(B, T, DIM)


# speedup vs baseline: 3.2396x; 1.0409x over previous
import functools

import jax
import jax.numpy as jnp
from jax.experimental import pallas as pl
from jax.experimental.pallas import tpu as pltpu

B, T, DIM = 1, 2048, 768
HEADS, DHEAD = 12, 64
HDIM = HEADS * DHEAD
CBS, SBS = 4, 4
NSEL, NMEM = 4, 1
WINDOW = 32
SCALE = 0.12
CDIM = CBS * DHEAD
HID = CDIM * 4
NBLK = T // CBS

TQ = 256
TK = 256
QT = T // TQ
CKP = 640
NEG = -1e30

f32 = jnp.float32


def _dot(a, b, precision=jax.lax.Precision.DEFAULT):
    return jax.lax.dot_general(a, b, (((1,), (0,)), ((), ())),
                               preferred_element_type=f32,
                               precision=precision)


def _dot_nt(a, b, precision=jax.lax.Precision.DEFAULT):
    return jax.lax.dot_general(a, b, (((1,), (1,)), ((), ())),
                               preferred_element_type=f32,
                               precision=precision)


def _pairswap(x):
    even = jax.lax.broadcasted_iota(jnp.int32, x.shape, 1) % 2 == 0
    return jnp.where(even, jnp.roll(x, -1, axis=1), jnp.roll(x, 1, axis=1))


def _qkv_kernel(x_ref, w3_ref, cos_ref, sin_ref,
                q_ref, k_ref, kb_ref, vb_ref):
    h = pl.program_id(1)
    xb = x_ref[...]
    z = _dot(xb, w3_ref[h])
    qh = z[:, 0:DHEAD]
    kh = z[:, DHEAD:2 * DHEAD]
    vh = z[:, 2 * DHEAD:3 * DHEAD]
    cos = cos_ref[...]
    sin = sin_ref[...]
    qr = qh * cos + _pairswap(qh) * sin
    kr = kh * cos + _pairswap(kh) * sin
    q_ref[0] = qr.astype(jnp.bfloat16)
    k_ref[0] = kr
    kb_ref[0] = kr.astype(jnp.bfloat16)
    vb_ref[0] = vh.astype(jnp.bfloat16)


def _cmlp_kernel(k2_ref, v2_ref, kp_ref, vp_ref,
                 kfc_ref, kpj_ref, vfc_ref, vpj_ref, ck_ref, cv_ref):
    kin = k2_ref[0] + kp_ref[0]
    vin = v2_ref[0] + vp_ref[0]
    hk = jnp.square(jax.nn.relu(_dot(kin, kfc_ref[...])))
    ck_ref[0] = _dot(hk, kpj_ref[...]).astype(jnp.bfloat16)
    hv = jnp.square(jax.nn.relu(_dot(vin, vfc_ref[...])))
    cv_ref[0] = _dot(hv, vpj_ref[...]).astype(jnp.bfloat16)


def _cattn_kernel(q_ref, ck_ref, cv_ref, cmask_ref, cout_ref, imp_ref):
    h = pl.program_id(1)
    qb = q_ref[0]
    sim = _dot_nt(qb, ck_ref[h]) * SCALE
    sim = jnp.where(cmask_ref[...] != 0, sim, NEG)
    m = jnp.max(sim, axis=1, keepdims=True)
    e = jnp.exp(sim - m)
    attn = e / jnp.sum(e, axis=1, keepdims=True)
    cout_ref[0] = _dot(attn, cv_ref[h])

    @pl.when(h == 0)
    def _():
        imp_ref[...] = jnp.zeros_like(imp_ref)
    imp_ref[...] += attn[:, :NBLK] * (1.0 / HEADS)


def _topk_kernel(imp_ref, emat_ref, sel_ref):
    val = imp_ref[...]
    iota = jax.lax.broadcasted_iota(jnp.int32, (TQ, NBLK), 1)
    sel = jnp.zeros((TQ, NBLK), f32)
    for _ in range(NSEL):
        m = jnp.max(val, axis=1, keepdims=True)
        cand = jnp.where(val == m, iota, NBLK * 4)
        idx = jnp.min(cand, axis=1, keepdims=True)
        oh = iota == idx
        sel = sel + oh.astype(f32)
        val = jnp.where(oh, -1.0, val)
    sel_ref[...] = (_dot(sel, emat_ref[...]) > 0.5).astype(jnp.int8)


WPAD = 32
WCOLS = 320


def _fw_kernel(q_ref, kw_ref, vw_ref, sel_ref, cd_ref, ca_ref, w0_ref,
               w1_ref, fout_ref, sout_ref):
    qt = pl.program_id(0)
    h = pl.program_id(1)
    qb = q_ref[0]

    def body(kt, carry):
        mf, lf, af = carry
        kb = kw_ref[h, pl.ds(kt * TK + WPAD, TK), :]
        vb = vw_ref[h, pl.ds(kt * TK + WPAD, TK), :]
        sim = _dot_nt(qb, kb) * SCALE
        fs = jnp.where(sel_ref[:, pl.ds(kt * TK, TK)] != 0, sim, NEG)
        mf2 = jnp.maximum(mf, jnp.max(fs, axis=1, keepdims=True))
        a = jnp.exp(mf - mf2)
        p = jnp.exp(fs - mf2)
        lf = lf * a + jnp.sum(p, axis=1, keepdims=True)
        af = af * a + _dot(p, vb)
        return mf2, lf, af

    init = (jnp.full((TQ, 1), NEG, f32), jnp.zeros((TQ, 1), f32),
            jnp.zeros((TQ, DHEAD), f32))
    mf, lf, af = jax.lax.fori_loop(0, qt, body, init)

    kb = kw_ref[h, pl.ds(qt * TK + WPAD, TK), :]
    vb = vw_ref[h, pl.ds(qt * TK + WPAD, TK), :]
    sim = _dot_nt(qb, kb) * SCALE
    dmask = (cd_ref[...] != 0) | ((ca_ref[...] != 0) &
                                  (sel_ref[:, pl.ds(qt * TK, TK)] != 0))
    fs = jnp.where(dmask, sim, NEG)
    mf2 = jnp.maximum(mf, jnp.max(fs, axis=1, keepdims=True))
    a = jnp.exp(mf - mf2)
    p = jnp.exp(fs - mf2)
    lf = lf * a + jnp.sum(p, axis=1, keepdims=True)
    af = af * a + _dot(p, vb)
    fout_ref[0] = af / lf

    kb = kw_ref[h, pl.ds(qt * TK, WCOLS), :]
    vb = vw_ref[h, pl.ds(qt * TK, WCOLS), :]
    ws = _dot_nt(qb, kb) * SCALE
    wm = jnp.where(qt == 0, w0_ref[...], w1_ref[...])
    ws = jnp.where(wm != 0, ws, NEG)
    mw = jnp.max(ws, axis=1, keepdims=True)
    pw = jnp.exp(ws - mw)
    sout_ref[0] = _dot(pw, vb) / jnp.sum(pw, axis=1, keepdims=True)


def _combine_kernel(x_ref, wg_ref, bg_ref, cout_ref, fout_ref, sout_ref,
                    cw_ref, out_ref):
    xb = x_ref[...]
    g = jax.nn.sigmoid(_dot(xb, wg_ref[...]) + bg_ref[...])
    acc = jnp.zeros((TQ, DIM), f32)
    for h in range(HEADS):
        gc = g[:, 3 * h:3 * h + 1]
        gf = g[:, 3 * h + 1:3 * h + 2]
        gs = g[:, 3 * h + 2:3 * h + 3]
        y = gc * cout_ref[h] + gf * fout_ref[h] + gs * sout_ref[h]
        acc = acc + _dot(y, cw_ref[h])
    out_ref[...] = acc


def _build_tables():
    i8 = jnp.int8
    inv = 1.0 / (10000.0 ** (jnp.arange(0, DHEAD, 2, dtype=f32) / DHEAD))
    freqs = jnp.arange(T, dtype=f32)[:, None] * inv[None, :]
    c = jnp.cos(freqs)
    si = jnp.sin(freqs)
    cos = jnp.stack([c, c], axis=-1).reshape(T, DHEAD)
    sin = jnp.stack([-si, si], axis=-1).reshape(T, DHEAD)
    emat = (jnp.arange(NBLK)[:, None] ==
            (jnp.arange(T)[None, :] // CBS)).astype(f32)
    tq = jnp.arange(T)[:, None]
    sc = jnp.arange(CKP)[None, :]
    cmask = (((sc < NBLK) & (CBS * sc + (CBS - 1) < tq)) |
             (sc == NBLK)).astype(i8)
    r = jnp.arange(TQ)[:, None]
    cc = jnp.arange(TK)[None, :]
    ca = (cc <= r)
    cd = (ca & ((r // SBS) == (cc // SBS))).astype(i8)
    ca = ca.astype(i8)
    cw = jnp.arange(WCOLS)[None, :]
    dt = r + WPAD - cw
    wbase = (dt >= 0) & (dt < WINDOW)
    w1 = wbase.astype(i8)
    w0 = (wbase & (cw >= WPAD)).astype(i8)
    return cos, sin, emat, cmask, cd, ca, w0, w1


@functools.partial(jax.jit, static_argnums=())
def kernel(x, qkv_w, k_fc_w, k_proj_w, v_fc_w, v_proj_w, compress_mem_kv,
           k_pos, v_pos, strat_w, strat_b, combine_w):
    x2 = x.reshape(T, DIM)
    cos, sin, emat, cmask, cd, ca, w0, w1 = _build_tables()

    w3 = jnp.transpose(qkv_w.reshape(3, HEADS, DHEAD, DIM), (1, 3, 0, 2))
    w3 = w3.reshape(HEADS, DIM, 3 * DHEAD)
    q, k, kb16, vb16 = pl.pallas_call(
        _qkv_kernel,
        grid=(QT, HEADS),
        in_specs=[
            pl.BlockSpec((TQ, DIM), lambda qt, h: (qt, 0)),
            pl.BlockSpec((HEADS, DIM, 3 * DHEAD), lambda qt, h: (0, 0, 0)),
            pl.BlockSpec((TQ, DHEAD), lambda qt, h: (qt, 0)),
            pl.BlockSpec((TQ, DHEAD), lambda qt, h: (qt, 0)),
        ],
        out_specs=[
            pl.BlockSpec((1, TQ, DHEAD), lambda qt, h: (h, qt, 0)),
            pl.BlockSpec((1, TQ, DHEAD), lambda qt, h: (h, qt, 0)),
            pl.BlockSpec((1, TQ, DHEAD), lambda qt, h: (h, qt, 0)),
            pl.BlockSpec((1, TQ, DHEAD), lambda qt, h: (h, qt, 0)),
        ],
        out_shape=[
            jax.ShapeDtypeStruct((HEADS, T, DHEAD), jnp.bfloat16),
            jax.ShapeDtypeStruct((HEADS, T, DHEAD), f32),
            jax.ShapeDtypeStruct((HEADS, T, DHEAD), jnp.bfloat16),
            jax.ShapeDtypeStruct((HEADS, T, DHEAD), jnp.bfloat16),
        ],
    )(x2, w3, cos, sin)

    k2 = k.reshape(HEADS, NBLK, CDIM)
    v2 = vb16.reshape(HEADS, NBLK, CDIM)
    kp = k_pos.reshape(HEADS, 1, CDIM)
    vp = v_pos.reshape(HEADS, 1, CDIM)
    TB = 128
    ck, cv = pl.pallas_call(
        _cmlp_kernel,
        grid=(HEADS, NBLK // TB),
        in_specs=[
            pl.BlockSpec((1, TB, CDIM), lambda h, b: (h, b, 0)),
            pl.BlockSpec((1, TB, CDIM), lambda h, b: (h, b, 0)),
            pl.BlockSpec((1, 1, CDIM), lambda h, b: (h, 0, 0)),
            pl.BlockSpec((1, 1, CDIM), lambda h, b: (h, 0, 0)),
            pl.BlockSpec((CDIM, HID), lambda h, b: (0, 0)),
            pl.BlockSpec((HID, DHEAD), lambda h, b: (0, 0)),
            pl.BlockSpec((CDIM, HID), lambda h, b: (0, 0)),
            pl.BlockSpec((HID, DHEAD), lambda h, b: (0, 0)),
        ],
        out_specs=[
            pl.BlockSpec((1, TB, DHEAD), lambda h, b: (h, b, 0)),
            pl.BlockSpec((1, TB, DHEAD), lambda h, b: (h, b, 0)),
        ],
        out_shape=[jax.ShapeDtypeStruct((HEADS, NBLK, DHEAD),
                                        jnp.bfloat16)] * 2,
    )(k2, v2, kp, vp, k_fc_w.T, k_proj_w.T, v_fc_w.T, v_proj_w.T)

    bf16 = jnp.bfloat16
    mem_k = compress_mem_kv[0].astype(bf16)
    mem_v = compress_mem_kv[1].astype(bf16)
    zpad = jnp.zeros((HEADS, CKP - NBLK - NMEM, DHEAD), bf16)
    ck_full = jnp.concatenate([ck, mem_k, zpad], axis=1)
    cv_full = jnp.concatenate([cv, mem_v, zpad], axis=1)
    cout, imp = pl.pallas_call(
        _cattn_kernel,
        grid=(QT, HEADS),
        in_specs=[
            pl.BlockSpec((1, TQ, DHEAD), lambda qt, h: (h, qt, 0)),
            pl.BlockSpec((HEADS, CKP, DHEAD), lambda qt, h: (0, 0, 0)),
            pl.BlockSpec((HEADS, CKP, DHEAD), lambda qt, h: (0, 0, 0)),
            pl.BlockSpec((TQ, CKP), lambda qt, h: (qt, 0)),
        ],
        out_specs=[
            pl.BlockSpec((1, TQ, DHEAD), lambda qt, h: (h, qt, 0)),
            pl.BlockSpec((TQ, NBLK), lambda qt, h: (qt, 0)),
        ],
        out_shape=[
            jax.ShapeDtypeStruct((HEADS, T, DHEAD), f32),
            jax.ShapeDtypeStruct((T, NBLK), f32),
        ],
    )(q, ck_full, cv_full, cmask)

    is_sel = pl.pallas_call(
        _topk_kernel,
        grid=(QT,),
        in_specs=[
            pl.BlockSpec((TQ, NBLK), lambda qt: (qt, 0)),
            pl.BlockSpec((NBLK, T), lambda qt: (0, 0)),
        ],
        out_specs=pl.BlockSpec((TQ, T), lambda qt: (qt, 0)),
        out_shape=jax.ShapeDtypeStruct((T, T), jnp.int8),
    )(imp, emat)

    TW = WPAD + T + (WCOLS - TK - WPAD)
    kw = jnp.concatenate(
        [jnp.zeros((HEADS, WPAD, DHEAD), bf16), kb16,
         jnp.zeros((HEADS, TW - WPAD - T, DHEAD), bf16)], axis=1)
    vw = jnp.concatenate(
        [jnp.zeros((HEADS, WPAD, DHEAD), bf16), vb16,
         jnp.zeros((HEADS, TW - WPAD - T, DHEAD), bf16)], axis=1)
    fout, sout = pl.pallas_call(
        _fw_kernel,
        grid=(QT, HEADS),
        in_specs=[
            pl.BlockSpec((1, TQ, DHEAD), lambda qt, h: (h, qt, 0)),
            pl.BlockSpec((HEADS, TW, DHEAD), lambda qt, h: (0, 0, 0)),
            pl.BlockSpec((HEADS, TW, DHEAD), lambda qt, h: (0, 0, 0)),
            pl.BlockSpec((TQ, T), lambda qt, h: (qt, 0)),
            pl.BlockSpec((TQ, TK), lambda qt, h: (0, 0)),
            pl.BlockSpec((TQ, TK), lambda qt, h: (0, 0)),
            pl.BlockSpec((TQ, WCOLS), lambda qt, h: (0, 0)),
            pl.BlockSpec((TQ, WCOLS), lambda qt, h: (0, 0)),
        ],
        out_specs=[
            pl.BlockSpec((1, TQ, DHEAD), lambda qt, h: (h, qt, 0)),
            pl.BlockSpec((1, TQ, DHEAD), lambda qt, h: (h, qt, 0)),
        ],
        out_shape=[jax.ShapeDtypeStruct((HEADS, T, DHEAD), f32)] * 2,
    )(q, kw, vw, is_sel, cd, ca, w0, w1)

    wg = jnp.concatenate([strat_w.T, jnp.zeros((DIM, 128 - 3 * HEADS), f32)],
                         axis=1)
    bg = jnp.concatenate([strat_b, jnp.zeros((128 - 3 * HEADS,), f32)])
    bg = bg.reshape(1, 128)
    cw = combine_w.T.reshape(HEADS, DHEAD, DIM)
    out = pl.pallas_call(
        _combine_kernel,
        grid=(QT,),
        in_specs=[
            pl.BlockSpec((TQ, DIM), lambda qt: (qt, 0)),
            pl.BlockSpec((DIM, 128), lambda qt: (0, 0)),
            pl.BlockSpec((1, 128), lambda qt: (0, 0)),
            pl.BlockSpec((HEADS, TQ, DHEAD), lambda qt: (0, qt, 0)),
            pl.BlockSpec((HEADS, TQ, DHEAD), lambda qt: (0, qt, 0)),
            pl.BlockSpec((HEADS, TQ, DHEAD), lambda qt: (0, qt, 0)),
            pl.BlockSpec((HEADS, DHEAD, DIM), lambda qt: (0, 0, 0)),
        ],
        out_specs=pl.BlockSpec((TQ, DIM), lambda qt: (qt, 0)),
        out_shape=jax.ShapeDtypeStruct((T, DIM), f32),
    )(x2, wg, bg, cout, fout, sout, cw)

    return out.reshape(B, T, DIM)


# speedup vs baseline: 3.3318x; 1.0284x over previous
import functools

import jax
import jax.numpy as jnp
from jax.experimental import pallas as pl
from jax.experimental.pallas import tpu as pltpu

B, T, DIM = 1, 2048, 768
HEADS, DHEAD = 12, 64
HDIM = HEADS * DHEAD
CBS, SBS = 4, 4
NSEL, NMEM = 4, 1
WINDOW = 32
SCALE = 0.12
CDIM = CBS * DHEAD
HID = CDIM * 4
NBLK = T // CBS

TQ = 256
TK = 256
QT = T // TQ
CKP = 640
NEG = -1e30

f32 = jnp.float32


def _dot(a, b, precision=jax.lax.Precision.DEFAULT):
    return jax.lax.dot_general(a, b, (((1,), (0,)), ((), ())),
                               preferred_element_type=f32,
                               precision=precision)


def _dot_nt(a, b, precision=jax.lax.Precision.DEFAULT):
    return jax.lax.dot_general(a, b, (((1,), (1,)), ((), ())),
                               preferred_element_type=f32,
                               precision=precision)


def _pairswap(x):
    even = jax.lax.broadcasted_iota(jnp.int32, x.shape, 1) % 2 == 0
    return jnp.where(even, jnp.roll(x, -1, axis=1), jnp.roll(x, 1, axis=1))


def _qkv_kernel(x_ref, w3_ref, cos_ref, sin_ref,
                q_ref, k_ref, kb_ref, vb_ref):
    hp = pl.program_id(1)
    xb = x_ref[...]
    z = _dot(xb, w3_ref[hp])
    cos = cos_ref[...]
    sin = sin_ref[...]
    for i in range(2):
        qh = z[:, 192 * i:192 * i + DHEAD]
        kh = z[:, 192 * i + DHEAD:192 * i + 2 * DHEAD]
        vh = z[:, 192 * i + 2 * DHEAD:192 * i + 3 * DHEAD]
        qr = qh * cos + _pairswap(qh) * sin
        kr = kh * cos + _pairswap(kh) * sin
        q_ref[i] = qr.astype(jnp.bfloat16)
        k_ref[i] = kr
        kb_ref[i] = kr.astype(jnp.bfloat16)
        vb_ref[i] = vh.astype(jnp.bfloat16)


def _cmlp_kernel(k2_ref, v2_ref, kp_ref, vp_ref,
                 kfc_ref, kpj_ref, vfc_ref, vpj_ref, ck_ref, cv_ref):
    kin = k2_ref[0] + kp_ref[0]
    vin = v2_ref[0] + vp_ref[0]
    hk = jnp.square(jax.nn.relu(_dot(kin, kfc_ref[...])))
    ck_ref[0] = _dot(hk, kpj_ref[...]).astype(jnp.bfloat16)
    hv = jnp.square(jax.nn.relu(_dot(vin, vfc_ref[...])))
    cv_ref[0] = _dot(hv, vpj_ref[...]).astype(jnp.bfloat16)


def _cattn_kernel(q_ref, ck_ref, cv_ref, cmask_ref, emat_ref,
                  cout_ref, sel_ref, imp_ref):
    h = pl.program_id(1)
    qb = q_ref[0]
    sim = _dot_nt(qb, ck_ref[h]) * SCALE
    sim = jnp.where(cmask_ref[...] != 0, sim, NEG)
    m = jnp.max(sim, axis=1, keepdims=True)
    e = jnp.exp(sim - m)
    attn = e / jnp.sum(e, axis=1, keepdims=True)
    cout_ref[0] = _dot(attn, cv_ref[h])

    @pl.when(h == 0)
    def _():
        imp_ref[...] = jnp.zeros_like(imp_ref)
    imp_ref[...] += attn[:, :NBLK] * (1.0 / HEADS)

    @pl.when(h == HEADS - 1)
    def _():
        val = imp_ref[...]
        iota = jax.lax.broadcasted_iota(jnp.int32, (TQ, NBLK), 1)
        sel = jnp.zeros((TQ, NBLK), f32)
        for _ in range(NSEL):
            mx = jnp.max(val, axis=1, keepdims=True)
            cand = jnp.where(val == mx, iota, NBLK * 4)
            idx = jnp.min(cand, axis=1, keepdims=True)
            oh = iota == idx
            sel = sel + oh.astype(f32)
            val = jnp.where(oh, -1.0, val)
        sel_ref[...] = (_dot(sel, emat_ref[...]) > 0.5).astype(jnp.int8)


WPAD = 32
WCOLS = 320


def _fw_kernel(q_ref, kw_ref, vw_ref, sel_ref, cd_ref, ca_ref, w0_ref,
               w1_ref, fout_ref, sout_ref):
    qt = pl.program_id(0)
    h = pl.program_id(1)
    qb = q_ref[0]

    def body(kt, carry):
        mf, lf, af = carry
        kb = kw_ref[h, pl.ds(kt * TK + WPAD, TK), :]
        vb = vw_ref[h, pl.ds(kt * TK + WPAD, TK), :]
        sim = _dot_nt(qb, kb) * SCALE
        fs = jnp.where(sel_ref[:, pl.ds(kt * TK, TK)] != 0, sim, NEG)
        mf2 = jnp.maximum(mf, jnp.max(fs, axis=1, keepdims=True))
        a = jnp.exp(mf - mf2)
        p = jnp.exp(fs - mf2)
        lf = lf * a + jnp.sum(p, axis=1, keepdims=True)
        af = af * a + _dot(p, vb)
        return mf2, lf, af

    init = (jnp.full((TQ, 1), NEG, f32), jnp.zeros((TQ, 1), f32),
            jnp.zeros((TQ, DHEAD), f32))
    mf, lf, af = jax.lax.fori_loop(0, qt, body, init)

    kb = kw_ref[h, pl.ds(qt * TK + WPAD, TK), :]
    vb = vw_ref[h, pl.ds(qt * TK + WPAD, TK), :]
    sim = _dot_nt(qb, kb) * SCALE
    dmask = (cd_ref[...] != 0) | ((ca_ref[...] != 0) &
                                  (sel_ref[:, pl.ds(qt * TK, TK)] != 0))
    fs = jnp.where(dmask, sim, NEG)
    mf2 = jnp.maximum(mf, jnp.max(fs, axis=1, keepdims=True))
    a = jnp.exp(mf - mf2)
    p = jnp.exp(fs - mf2)
    lf = lf * a + jnp.sum(p, axis=1, keepdims=True)
    af = af * a + _dot(p, vb)
    fout_ref[0] = af / lf

    kb = kw_ref[h, pl.ds(qt * TK, WCOLS), :]
    vb = vw_ref[h, pl.ds(qt * TK, WCOLS), :]
    ws = _dot_nt(qb, kb) * SCALE
    wm = jnp.where(qt == 0, w0_ref[...], w1_ref[...])
    ws = jnp.where(wm != 0, ws, NEG)
    mw = jnp.max(ws, axis=1, keepdims=True)
    pw = jnp.exp(ws - mw)
    sout_ref[0] = _dot(pw, vb) / jnp.sum(pw, axis=1, keepdims=True)


def _combine_kernel(x_ref, wg_ref, bg_ref, cout_ref, fout_ref, sout_ref,
                    cw_ref, out_ref):
    xb = x_ref[...]
    g = jax.nn.sigmoid(_dot(xb, wg_ref[...]) + bg_ref[...])
    acc = jnp.zeros((TQ, DIM), f32)
    for h in range(HEADS):
        gc = g[:, 3 * h:3 * h + 1]
        gf = g[:, 3 * h + 1:3 * h + 2]
        gs = g[:, 3 * h + 2:3 * h + 3]
        y = gc * cout_ref[h] + gf * fout_ref[h] + gs * sout_ref[h]
        acc = acc + _dot(y, cw_ref[h])
    out_ref[...] = acc


def _build_tables():
    i8 = jnp.int8
    inv = 1.0 / (10000.0 ** (jnp.arange(0, DHEAD, 2, dtype=f32) / DHEAD))
    freqs = jnp.arange(T, dtype=f32)[:, None] * inv[None, :]
    c = jnp.cos(freqs)
    si = jnp.sin(freqs)
    cos = jnp.stack([c, c], axis=-1).reshape(T, DHEAD)
    sin = jnp.stack([-si, si], axis=-1).reshape(T, DHEAD)
    emat = (jnp.arange(NBLK)[:, None] ==
            (jnp.arange(T)[None, :] // CBS)).astype(f32)
    tq = jnp.arange(T)[:, None]
    sc = jnp.arange(CKP)[None, :]
    cmask = (((sc < NBLK) & (CBS * sc + (CBS - 1) < tq)) |
             (sc == NBLK)).astype(i8)
    r = jnp.arange(TQ)[:, None]
    cc = jnp.arange(TK)[None, :]
    ca = (cc <= r)
    cd = (ca & ((r // SBS) == (cc // SBS))).astype(i8)
    ca = ca.astype(i8)
    cw = jnp.arange(WCOLS)[None, :]
    dt = r + WPAD - cw
    wbase = (dt >= 0) & (dt < WINDOW)
    w1 = wbase.astype(i8)
    w0 = (wbase & (cw >= WPAD)).astype(i8)
    return cos, sin, emat, cmask, cd, ca, w0, w1


@functools.partial(jax.jit, static_argnums=())
def kernel(x, qkv_w, k_fc_w, k_proj_w, v_fc_w, v_proj_w, compress_mem_kv,
           k_pos, v_pos, strat_w, strat_b, combine_w):
    x2 = x.reshape(T, DIM)
    cos, sin, emat, cmask, cd, ca, w0, w1 = _build_tables()

    w3 = jnp.transpose(qkv_w.reshape(3, HEADS, DHEAD, DIM), (1, 3, 0, 2))
    w3 = w3.reshape(HEADS // 2, 2, DIM, 3 * DHEAD)
    w3 = w3.transpose(0, 2, 1, 3).reshape(HEADS // 2, DIM, 6 * DHEAD)
    q, k, kb16, vb16 = pl.pallas_call(
        _qkv_kernel,
        grid=(QT, HEADS // 2),
        in_specs=[
            pl.BlockSpec((TQ, DIM), lambda qt, h: (qt, 0)),
            pl.BlockSpec((HEADS // 2, DIM, 6 * DHEAD),
                         lambda qt, h: (0, 0, 0)),
            pl.BlockSpec((TQ, DHEAD), lambda qt, h: (qt, 0)),
            pl.BlockSpec((TQ, DHEAD), lambda qt, h: (qt, 0)),
        ],
        out_specs=[
            pl.BlockSpec((2, TQ, DHEAD), lambda qt, h: (h, qt, 0)),
            pl.BlockSpec((2, TQ, DHEAD), lambda qt, h: (h, qt, 0)),
            pl.BlockSpec((2, TQ, DHEAD), lambda qt, h: (h, qt, 0)),
            pl.BlockSpec((2, TQ, DHEAD), lambda qt, h: (h, qt, 0)),
        ],
        out_shape=[
            jax.ShapeDtypeStruct((HEADS, T, DHEAD), jnp.bfloat16),
            jax.ShapeDtypeStruct((HEADS, T, DHEAD), f32),
            jax.ShapeDtypeStruct((HEADS, T, DHEAD), jnp.bfloat16),
            jax.ShapeDtypeStruct((HEADS, T, DHEAD), jnp.bfloat16),
        ],
    )(x2, w3, cos, sin)

    k2 = k.reshape(HEADS, NBLK, CDIM)
    v2 = vb16.reshape(HEADS, NBLK, CDIM)
    kp = k_pos.reshape(HEADS, 1, CDIM)
    vp = v_pos.reshape(HEADS, 1, CDIM)
    TB = 128
    ck, cv = pl.pallas_call(
        _cmlp_kernel,
        grid=(HEADS, NBLK // TB),
        in_specs=[
            pl.BlockSpec((1, TB, CDIM), lambda h, b: (h, b, 0)),
            pl.BlockSpec((1, TB, CDIM), lambda h, b: (h, b, 0)),
            pl.BlockSpec((1, 1, CDIM), lambda h, b: (h, 0, 0)),
            pl.BlockSpec((1, 1, CDIM), lambda h, b: (h, 0, 0)),
            pl.BlockSpec((CDIM, HID), lambda h, b: (0, 0)),
            pl.BlockSpec((HID, DHEAD), lambda h, b: (0, 0)),
            pl.BlockSpec((CDIM, HID), lambda h, b: (0, 0)),
            pl.BlockSpec((HID, DHEAD), lambda h, b: (0, 0)),
        ],
        out_specs=[
            pl.BlockSpec((1, TB, DHEAD), lambda h, b: (h, b, 0)),
            pl.BlockSpec((1, TB, DHEAD), lambda h, b: (h, b, 0)),
        ],
        out_shape=[jax.ShapeDtypeStruct((HEADS, NBLK, DHEAD),
                                        jnp.bfloat16)] * 2,
    )(k2, v2, kp, vp, k_fc_w.T, k_proj_w.T, v_fc_w.T, v_proj_w.T)

    bf16 = jnp.bfloat16
    mem_k = compress_mem_kv[0].astype(bf16)
    mem_v = compress_mem_kv[1].astype(bf16)
    zpad = jnp.zeros((HEADS, CKP - NBLK - NMEM, DHEAD), bf16)
    ck_full = jnp.concatenate([ck, mem_k, zpad], axis=1)
    cv_full = jnp.concatenate([cv, mem_v, zpad], axis=1)
    cout, is_sel = pl.pallas_call(
        _cattn_kernel,
        grid=(QT, HEADS),
        in_specs=[
            pl.BlockSpec((1, TQ, DHEAD), lambda qt, h: (h, qt, 0)),
            pl.BlockSpec((HEADS, CKP, DHEAD), lambda qt, h: (0, 0, 0)),
            pl.BlockSpec((HEADS, CKP, DHEAD), lambda qt, h: (0, 0, 0)),
            pl.BlockSpec((TQ, CKP), lambda qt, h: (qt, 0)),
            pl.BlockSpec((NBLK, T), lambda qt, h: (0, 0)),
        ],
        out_specs=[
            pl.BlockSpec((1, TQ, DHEAD), lambda qt, h: (h, qt, 0)),
            pl.BlockSpec((TQ, T), lambda qt, h: (qt, 0)),
        ],
        out_shape=[
            jax.ShapeDtypeStruct((HEADS, T, DHEAD), f32),
            jax.ShapeDtypeStruct((T, T), jnp.int8),
        ],
        scratch_shapes=[pltpu.VMEM((TQ, NBLK), f32)],
    )(q, ck_full, cv_full, cmask, emat)

    TW = WPAD + T + (WCOLS - TK - WPAD)
    kw = jnp.concatenate(
        [jnp.zeros((HEADS, WPAD, DHEAD), bf16), kb16,
         jnp.zeros((HEADS, TW - WPAD - T, DHEAD), bf16)], axis=1)
    vw = jnp.concatenate(
        [jnp.zeros((HEADS, WPAD, DHEAD), bf16), vb16,
         jnp.zeros((HEADS, TW - WPAD - T, DHEAD), bf16)], axis=1)
    fout, sout = pl.pallas_call(
        _fw_kernel,
        grid=(QT, HEADS),
        in_specs=[
            pl.BlockSpec((1, TQ, DHEAD), lambda qt, h: (h, qt, 0)),
            pl.BlockSpec((HEADS, TW, DHEAD), lambda qt, h: (0, 0, 0)),
            pl.BlockSpec((HEADS, TW, DHEAD), lambda qt, h: (0, 0, 0)),
            pl.BlockSpec((TQ, T), lambda qt, h: (qt, 0)),
            pl.BlockSpec((TQ, TK), lambda qt, h: (0, 0)),
            pl.BlockSpec((TQ, TK), lambda qt, h: (0, 0)),
            pl.BlockSpec((TQ, WCOLS), lambda qt, h: (0, 0)),
            pl.BlockSpec((TQ, WCOLS), lambda qt, h: (0, 0)),
        ],
        out_specs=[
            pl.BlockSpec((1, TQ, DHEAD), lambda qt, h: (h, qt, 0)),
            pl.BlockSpec((1, TQ, DHEAD), lambda qt, h: (h, qt, 0)),
        ],
        out_shape=[jax.ShapeDtypeStruct((HEADS, T, DHEAD), f32)] * 2,
    )(q, kw, vw, is_sel, cd, ca, w0, w1)

    wg = jnp.concatenate([strat_w.T, jnp.zeros((DIM, 128 - 3 * HEADS), f32)],
                         axis=1)
    bg = jnp.concatenate([strat_b, jnp.zeros((128 - 3 * HEADS,), f32)])
    bg = bg.reshape(1, 128)
    cw = combine_w.T.reshape(HEADS, DHEAD, DIM)
    out = pl.pallas_call(
        _combine_kernel,
        grid=(QT,),
        in_specs=[
            pl.BlockSpec((TQ, DIM), lambda qt: (qt, 0)),
            pl.BlockSpec((DIM, 128), lambda qt: (0, 0)),
            pl.BlockSpec((1, 128), lambda qt: (0, 0)),
            pl.BlockSpec((HEADS, TQ, DHEAD), lambda qt: (0, qt, 0)),
            pl.BlockSpec((HEADS, TQ, DHEAD), lambda qt: (0, qt, 0)),
            pl.BlockSpec((HEADS, TQ, DHEAD), lambda qt: (0, qt, 0)),
            pl.BlockSpec((HEADS, DHEAD, DIM), lambda qt: (0, 0, 0)),
        ],
        out_specs=pl.BlockSpec((TQ, DIM), lambda qt: (qt, 0)),
        out_shape=jax.ShapeDtypeStruct((T, DIM), f32),
    )(x2, wg, bg, cout, fout, sout, cw)

    return out.reshape(B, T, DIM)
